# baseline (speedup 1.0000x reference)
import functools

import jax
import jax.numpy as jnp
from jax import lax
from jax.experimental import pallas as pl
from jax.experimental.pallas import tpu as pltpu
from jax.experimental.pallas import tpu_sc as plsc

TM = 128
BN = 512


def _gelu(x):
    return x * 0.5 * (1.0 + jax.lax.erf(x * 0.7071067811865476))


def _router_kernel(x_ref, wg_ref, g0_ref, g1_ref, c0_ref, c1_ref,
                   cnt_ref, meta_ref, carry_s,
                   *, bn, nb, tm, nt, n_experts):
    b = pl.program_id(0)

    @pl.when(b == 0)
    def _init():
        carry_s[...] = jnp.zeros_like(carry_s)

    logits = jnp.dot(x_ref[...], wg_ref[...],
                     preferred_element_type=jnp.float32)
    eids = jax.lax.broadcasted_iota(jnp.int32, logits.shape, 1)
    top1 = jnp.max(logits, axis=-1, keepdims=True)
    a1 = jnp.argmax(logits, axis=-1)[:, None]
    masked = jnp.where(eids == a1, -jnp.inf, logits)
    top2 = jnp.max(masked, axis=-1, keepdims=True)
    a2 = jnp.argmax(masked, axis=-1)[:, None]
    m = jnp.maximum(top1, top2)
    e1 = jnp.exp(top1 - m)
    e2 = jnp.exp(top2 - m)
    z = e1 + e2
    g0_ref[...] = e1 / z
    g1_ref[...] = e2 / z

    amat = ((eids == a1) | (eids == a2)).astype(jnp.float32)
    ri = jax.lax.broadcasted_iota(jnp.int32, (bn, bn), 0)
    ci = jax.lax.broadcasted_iota(jnp.int32, (bn, bn), 1)
    tri = (ci < ri).astype(jnp.float32)
    rank_b = jnp.dot(tri, amat, preferred_element_type=jnp.float32) + carry_s[...]
    r1 = jnp.sum(jnp.where(eids == a1, rank_b, 0.0), axis=1, keepdims=True)
    r2 = jnp.sum(jnp.where(eids == a2, rank_b, 0.0), axis=1, keepdims=True)
    c0_ref[...] = a1 * 8192 + r1.astype(jnp.int32)
    c1_ref[...] = a2 * 8192 + r2.astype(jnp.int32)
    carry_s[...] += jnp.sum(amat, axis=0, keepdims=True)

    @pl.when(b == nb - 1)
    def _meta():
        counts = carry_s[...].astype(jnp.int32)
        cids = jax.lax.broadcasted_iota(jnp.int32, cnt_ref.shape, 0)
        pb_acc = jnp.zeros(cnt_ref.shape, jnp.int32)
        iota_t = jax.lax.broadcasted_iota(jnp.int32, (1, nt), 1) * tm
        te_acc = jnp.zeros((1, nt), jnp.int32)
        s = jnp.zeros((), jnp.int32)
        pcs = []
        for e in range(n_experts):
            ne = counts[0, e]
            pb_acc = pb_acc + jnp.where(cids == e, s, 0)
            pc = ((ne + tm - 1) // tm) * tm
            pcs.append(pc)
            s = s + pc
            te_acc = te_acc + (iota_t >= s).astype(jnp.int32)
        cnt_ref[...] = pb_acc

        last_used = jnp.zeros((), jnp.int32)
        for e in range(n_experts):
            last_used = jnp.where(pcs[e] > 0, e, last_used)
        te_vals = jnp.minimum(te_acc, last_used)
        tv = (iota_t < s).astype(jnp.int32)

        rid = jnp.zeros((), jnp.int32)
        run_par = []
        for e in range(n_experts):
            run_par.append(rid % 2)
            rid = rid + (pcs[e] > 0).astype(jnp.int32)
        nxt = jnp.full((), -1, jnp.int32)
        nxt_list = [None] * n_experts
        for e in reversed(range(n_experts)):
            nxt_list[e] = nxt
            nxt = jnp.where(pcs[e] > 0, e, nxt)
        par_t = jnp.zeros((1, nt), jnp.int32)
        nx_t = jnp.zeros((1, nt), jnp.int32)
        for e in range(n_experts):
            par_t = jnp.where(te_vals == e, run_par[e], par_t)
            nx_t = jnp.where(te_vals == e, nxt_list[e], nx_t)
        meta_ref[...] = jnp.concatenate([te_vals, tv, par_t, nx_t], axis=1)


def _router(xf, Wg, nt):
    n, c = xf.shape
    e = Wg.shape[1]
    nb = n // BN
    return pl.pallas_call(
        functools.partial(_router_kernel, bn=BN, nb=nb, tm=TM, nt=nt,
                          n_experts=e),
        grid=(nb,),
        in_specs=[
            pl.BlockSpec((BN, c), lambda b: (b, 0)),
            pl.BlockSpec((c, e), lambda b: (0, 0)),
        ],
        out_specs=[
            pl.BlockSpec((BN, 1), lambda b: (b, 0)),
            pl.BlockSpec((BN, 1), lambda b: (b, 0)),
            pl.BlockSpec((BN, 1), lambda b: (b, 0)),
            pl.BlockSpec((BN, 1), lambda b: (b, 0)),
            pl.BlockSpec((8, 16), lambda b: (0, 0)),
            pl.BlockSpec((1, 4 * nt), lambda b: (0, 0)),
        ],
        out_shape=[
            jax.ShapeDtypeStruct((n, 1), jnp.float32),
            jax.ShapeDtypeStruct((n, 1), jnp.float32),
            jax.ShapeDtypeStruct((n, 1), jnp.int32),
            jax.ShapeDtypeStruct((n, 1), jnp.int32),
            jax.ShapeDtypeStruct((8, 16), jnp.int32),
            jax.ShapeDtypeStruct((1, 4 * nt), jnp.int32),
        ],
        scratch_shapes=[pltpu.VMEM((1, e), jnp.float32)],
    )(xf, Wg)


def _slot_prelude(pb_hbm, pb_ref):
    pltpu.sync_copy(pb_hbm, pb_ref)


def _slot_chunk(code_hbm, base, ch, n_experts, c_v, pb_ref, idx_v):
    pltpu.sync_copy(code_hbm.at[pl.ds(base, ch)], c_v)
    for j in range(ch // 16):
        sl = pl.ds(j * 16, 16)
        c16 = c_v[sl]
        e16 = jnp.right_shift(c16, 13)
        acc = c16 & 8191
        for e in range(n_experts):
            acc = acc + jnp.where(e16 == e, pb_ref[e], 0)
        idx_v[sl] = acc


def _dispatch_body(tpw, ch, x_hbm, c0_hbm, c1_hbm,
                   pb_hbm, xs_out, rows_v, idx0_v, idx1_v, c_v, pb_ref,
                   sem):
    wid = lax.axis_index("s") * 2 + lax.axis_index("c")
    _slot_prelude(pb_hbm, pb_ref)
    for c in range(tpw // ch):
        base = pl.multiple_of(wid * tpw + c * ch, ch)
        pltpu.sync_copy(x_hbm.at[pl.ds(base, ch)], rows_v)
        _slot_chunk(c0_hbm, base, ch, 8, c_v, pb_ref, idx0_v)
        cp0 = pltpu.async_copy(rows_v, xs_out.at[idx0_v], sem)
        _slot_chunk(c1_hbm, base, ch, 8, c_v, pb_ref, idx1_v)
        cp1 = pltpu.async_copy(rows_v, xs_out.at[idx1_v], sem)
        cp0.wait()
        cp1.wait()


def _dispatch(xf, c0, c1, pb16, nslot):
    n, c = xf.shape
    nw = 32
    tpw = n // nw
    ch = min(128, tpw)
    mesh = plsc.VectorSubcoreMesh(core_axis_name="c", subcore_axis_name="s")
    f = pl.kernel(
        functools.partial(_dispatch_body, tpw, ch),
        mesh=mesh,
        out_type=jax.ShapeDtypeStruct((nslot, c), jnp.float32),
        scratch_types=[
            pltpu.VMEM((ch, c), jnp.float32),
            pltpu.VMEM((ch,), jnp.int32),
            pltpu.VMEM((ch,), jnp.int32),
            pltpu.VMEM((ch,), jnp.int32),
            pltpu.VMEM((8, 16), jnp.int32),
            pltpu.SemaphoreType.DMA,
        ],
    )
    return f(xf, c0, c1, pb16)


def _gemm_kernel(meta_ref, xs_ref, b1_ref, b2_ref, w1_hbm, w2_hbm, out_ref,
                 w1_buf, w2_buf, sem1, sem2, *, nt):
    i = pl.program_id(0)
    e = meta_ref[i]
    valid = meta_ref[nt + i]
    par = meta_ref[2 * nt + i]
    nxt = meta_ref[3 * nt + i]
    prev = meta_ref[jnp.maximum(i - 1, 0)]
    first = jnp.logical_or(i == 0, prev != e)

    def _issue(expert, slot):
        pltpu.make_async_copy(w1_hbm.at[expert], w1_buf.at[slot],
                              sem1.at[slot]).start()
        pltpu.make_async_copy(w2_hbm.at[expert], w2_buf.at[slot],
                              sem2.at[slot]).start()

    @pl.when(i == 0)
    def _prologue():
        _issue(e, par)

    @pl.when(first)
    def _run_start():
        @pl.when(nxt >= 0)
        def _prefetch_next():
            _issue(nxt, 1 - par)

        pltpu.make_async_copy(w1_hbm.at[e], w1_buf.at[par],
                              sem1.at[par]).wait()
        pltpu.make_async_copy(w2_hbm.at[e], w2_buf.at[par],
                              sem2.at[par]).wait()

    @pl.when(valid == 1)
    def _compute():
        xb = xs_ref[...].astype(jnp.bfloat16)
        w1 = w1_buf[par].astype(jnp.bfloat16)
        h = _gelu(jnp.dot(xb, w1, preferred_element_type=jnp.float32)
                  + b1_ref[0])
        w2 = w2_buf[par].astype(jnp.bfloat16)
        out_ref[...] = (jnp.dot(h.astype(jnp.bfloat16), w2,
                                preferred_element_type=jnp.float32)
                        + b2_ref[0])


def _grouped_gemm(meta, xs, W1, b1, W2, b2, nt):
    nslot, c = xs.shape
    e, _, h = W1.shape
    grid_spec = pltpu.PrefetchScalarGridSpec(
        num_scalar_prefetch=1,
        grid=(nt,),
        in_specs=[
            pl.BlockSpec((TM, c), lambda i, m: (i, 0)),
            pl.BlockSpec((1, 1, h), lambda i, m: (m[i], 0, 0)),
            pl.BlockSpec((1, 1, c), lambda i, m: (m[i], 0, 0)),
            pl.BlockSpec(memory_space=pl.ANY),
            pl.BlockSpec(memory_space=pl.ANY),
        ],
        out_specs=pl.BlockSpec((TM, c), lambda i, m: (i, 0)),
        scratch_shapes=[
            pltpu.VMEM((2, c, h), jnp.float32),
            pltpu.VMEM((2, h, c), jnp.float32),
            pltpu.SemaphoreType.DMA((2,)),
            pltpu.SemaphoreType.DMA((2,)),
        ],
    )
    return pl.pallas_call(
        functools.partial(_gemm_kernel, nt=nt),
        grid_spec=grid_spec,
        out_shape=jax.ShapeDtypeStruct((nslot, c), jnp.float32),
        compiler_params=pltpu.CompilerParams(
            vmem_limit_bytes=100 * 1024 * 1024),
    )(meta, xs, b1.reshape(e, 1, h), b2.reshape(e, 1, c), W1, W2)


def _gather2_body(tpw, ch, ys_hbm, c0_hbm, c1_hbm,
                  pb_hbm, z0_hbm, z1_hbm, i0_v, i1_v, y0_v, y1_v, c_v, pb_ref,
                  sem):
    wid = lax.axis_index("s") * 2 + lax.axis_index("c")
    _slot_prelude(pb_hbm, pb_ref)
    for c in range(tpw // ch):
        base = pl.multiple_of(wid * tpw + c * ch, ch)
        _slot_chunk(c0_hbm, base, ch, 8, c_v, pb_ref, i0_v)
        cp0 = pltpu.async_copy(ys_hbm.at[i0_v], y0_v, sem)
        _slot_chunk(c1_hbm, base, ch, 8, c_v, pb_ref, i1_v)
        cp1 = pltpu.async_copy(ys_hbm.at[i1_v], y1_v, sem)
        cp0.wait()
        pltpu.sync_copy(y0_v, z0_hbm.at[pl.ds(base, ch)])
        cp1.wait()
        pltpu.sync_copy(y1_v, z1_hbm.at[pl.ds(base, ch)])


def _gather2(ys, c0, c1, pb16, n):
    nslot, c = ys.shape
    nw = 32
    tpw = n // nw
    ch = min(64, tpw)
    mesh = plsc.VectorSubcoreMesh(core_axis_name="c", subcore_axis_name="s")
    f = pl.kernel(
        functools.partial(_gather2_body, tpw, ch),
        mesh=mesh,
        out_type=(jax.ShapeDtypeStruct((n, c), jnp.float32),
                  jax.ShapeDtypeStruct((n, c), jnp.float32)),
        scratch_types=[
            pltpu.VMEM((ch,), jnp.int32),
            pltpu.VMEM((ch,), jnp.int32),
            pltpu.VMEM((ch, c), jnp.float32),
            pltpu.VMEM((ch, c), jnp.float32),
            pltpu.VMEM((ch,), jnp.int32),
            pltpu.VMEM((8, 16), jnp.int32),
            pltpu.SemaphoreType.DMA,
        ],
    )
    return f(ys, c0, c1, pb16)


def _blend_kernel(z0_ref, z1_ref, g0_ref, g1_ref, out_ref):
    out_ref[...] = g0_ref[...] * z0_ref[...] + g1_ref[...] * z1_ref[...]


def _blend(z0, z1, g0, g1):
    n, c = z0.shape
    bn = min(n, 1024)
    return pl.pallas_call(
        _blend_kernel,
        grid=(n // bn,),
        in_specs=[
            pl.BlockSpec((bn, c), lambda b: (b, 0)),
            pl.BlockSpec((bn, c), lambda b: (b, 0)),
            pl.BlockSpec((bn, 1), lambda b: (b, 0)),
            pl.BlockSpec((bn, 1), lambda b: (b, 0)),
        ],
        out_specs=pl.BlockSpec((bn, c), lambda b: (b, 0)),
        out_shape=jax.ShapeDtypeStruct((n, c), jnp.float32),
    )(z0, z1, g0, g1)


def kernel(x, Wg, W1, b1, W2, b2):
    Bx, Tx, C = x.shape
    E = Wg.shape[1]
    N = Bx * Tx
    nt = (2 * N) // TM + E
    nslot = nt * TM
    xf = x.reshape(N, C)

    g0, g1, c0, c1, pb16, meta = _router(xf, Wg, nt)
    c0 = c0.reshape(N)
    c1 = c1.reshape(N)
    meta = meta.reshape(4 * nt)

    xs = _dispatch(xf, c0, c1, pb16, nslot)
    ys = _grouped_gemm(meta, xs, W1, b1, W2, b2, nt)
    z0, z1 = _gather2(ys, c0, c1, pb16, N)
    outf = _blend(z0, z1, g0, g1)
    return outf.reshape(Bx, Tx, C)


# speedup vs baseline: 1.1858x; 1.1858x over previous
import functools

import jax
import jax.numpy as jnp
from jax import lax
from jax.experimental import pallas as pl
from jax.experimental.pallas import tpu as pltpu
from jax.experimental.pallas import tpu_sc as plsc

TM = 256
BN = 512


def _gelu(x):
    return x * 0.5 * (1.0 + jax.lax.erf(x * 0.7071067811865476))


def _router_kernel(x_ref, wg_ref, g0_ref, g1_ref, c0_ref, c1_ref,
                   cnt_ref, meta_ref, carry_s,
                   *, bn, nb, tm, nt, n_experts):
    b = pl.program_id(0)

    @pl.when(b == 0)
    def _init():
        carry_s[...] = jnp.zeros_like(carry_s)

    logits = jnp.dot(x_ref[...], wg_ref[...],
                     preferred_element_type=jnp.float32)
    eids = jax.lax.broadcasted_iota(jnp.int32, logits.shape, 1)
    top1 = jnp.max(logits, axis=-1, keepdims=True)
    a1 = jnp.argmax(logits, axis=-1)[:, None]
    masked = jnp.where(eids == a1, -jnp.inf, logits)
    top2 = jnp.max(masked, axis=-1, keepdims=True)
    a2 = jnp.argmax(masked, axis=-1)[:, None]
    m = jnp.maximum(top1, top2)
    e1 = jnp.exp(top1 - m)
    e2 = jnp.exp(top2 - m)
    z = e1 + e2
    g0_ref[...] = e1 / z
    g1_ref[...] = e2 / z

    amat = ((eids == a1) | (eids == a2)).astype(jnp.float32)
    ri = jax.lax.broadcasted_iota(jnp.int32, (bn, bn), 0)
    ci = jax.lax.broadcasted_iota(jnp.int32, (bn, bn), 1)
    tri = (ci < ri).astype(jnp.float32)
    rank_b = jnp.dot(tri, amat, preferred_element_type=jnp.float32) + carry_s[...]
    r1 = jnp.sum(jnp.where(eids == a1, rank_b, 0.0), axis=1, keepdims=True)
    r2 = jnp.sum(jnp.where(eids == a2, rank_b, 0.0), axis=1, keepdims=True)
    c0_ref[...] = a1 * 8192 + r1.astype(jnp.int32)
    c1_ref[...] = a2 * 8192 + r2.astype(jnp.int32)
    carry_s[...] += jnp.sum(amat, axis=0, keepdims=True)

    @pl.when(b == nb - 1)
    def _meta():
        counts = carry_s[...].astype(jnp.int32)
        cids = jax.lax.broadcasted_iota(jnp.int32, cnt_ref.shape, 0)
        pb_acc = jnp.zeros(cnt_ref.shape, jnp.int32)
        iota_t = jax.lax.broadcasted_iota(jnp.int32, (1, nt), 1) * tm
        te_acc = jnp.zeros((1, nt), jnp.int32)
        s = jnp.zeros((), jnp.int32)
        pcs = []
        for e in range(n_experts):
            ne = counts[0, e]
            pb_acc = pb_acc + jnp.where(cids == e, s, 0)
            pc = ((ne + tm - 1) // tm) * tm
            pcs.append(pc)
            s = s + pc
            te_acc = te_acc + (iota_t >= s).astype(jnp.int32)
        cnt_ref[...] = pb_acc

        last_used = jnp.zeros((), jnp.int32)
        for e in range(n_experts):
            last_used = jnp.where(pcs[e] > 0, e, last_used)
        te_vals = jnp.minimum(te_acc, last_used)
        tv = (iota_t < s).astype(jnp.int32)

        rid = jnp.zeros((), jnp.int32)
        run_par = []
        for e in range(n_experts):
            run_par.append(rid % 2)
            rid = rid + (pcs[e] > 0).astype(jnp.int32)
        nxt = jnp.full((), -1, jnp.int32)
        nxt_list = [None] * n_experts
        for e in reversed(range(n_experts)):
            nxt_list[e] = nxt
            nxt = jnp.where(pcs[e] > 0, e, nxt)
        par_t = jnp.zeros((1, nt), jnp.int32)
        nx_t = jnp.zeros((1, nt), jnp.int32)
        for e in range(n_experts):
            par_t = jnp.where(te_vals == e, run_par[e], par_t)
            nx_t = jnp.where(te_vals == e, nxt_list[e], nx_t)
        meta_ref[...] = jnp.concatenate([te_vals, tv, par_t, nx_t], axis=1)


def _router(xf, Wg, nt):
    n, c = xf.shape
    e = Wg.shape[1]
    nb = n // BN
    return pl.pallas_call(
        functools.partial(_router_kernel, bn=BN, nb=nb, tm=TM, nt=nt,
                          n_experts=e),
        grid=(nb,),
        in_specs=[
            pl.BlockSpec((BN, c), lambda b: (b, 0)),
            pl.BlockSpec((c, e), lambda b: (0, 0)),
        ],
        out_specs=[
            pl.BlockSpec((BN, 1), lambda b: (b, 0)),
            pl.BlockSpec((BN, 1), lambda b: (b, 0)),
            pl.BlockSpec((BN, 1), lambda b: (b, 0)),
            pl.BlockSpec((BN, 1), lambda b: (b, 0)),
            pl.BlockSpec((8, 16), lambda b: (0, 0)),
            pl.BlockSpec((1, 4 * nt), lambda b: (0, 0)),
        ],
        out_shape=[
            jax.ShapeDtypeStruct((n, 1), jnp.float32),
            jax.ShapeDtypeStruct((n, 1), jnp.float32),
            jax.ShapeDtypeStruct((n, 1), jnp.int32),
            jax.ShapeDtypeStruct((n, 1), jnp.int32),
            jax.ShapeDtypeStruct((8, 16), jnp.int32),
            jax.ShapeDtypeStruct((1, 4 * nt), jnp.int32),
        ],
        scratch_shapes=[pltpu.VMEM((1, e), jnp.float32)],
    )(xf, Wg)


def _slot_prelude(pb_hbm, pb_ref):
    pltpu.sync_copy(pb_hbm, pb_ref)


def _slot_chunk(code_hbm, base, ch, n_experts, c_v, pb_ref, idx_v):
    pltpu.sync_copy(code_hbm.at[pl.ds(base, ch)], c_v)
    for j in range(ch // 16):
        sl = pl.ds(j * 16, 16)
        c16 = c_v[sl]
        e16 = jnp.right_shift(c16, 13)
        acc = c16 & 8191
        for e in range(n_experts):
            acc = acc + jnp.where(e16 == e, pb_ref[e], 0)
        idx_v[sl] = acc


def _dispatch_body(tpw, ch, x_hbm, c0_hbm, c1_hbm,
                   pb_hbm, xs_out, rows_v, idx0_v, idx1_v, c_v, pb_ref,
                   sem):
    wid = lax.axis_index("s") * 2 + lax.axis_index("c")
    _slot_prelude(pb_hbm, pb_ref)
    for c in range(tpw // ch):
        base = pl.multiple_of(wid * tpw + c * ch, ch)
        pltpu.sync_copy(x_hbm.at[pl.ds(base, ch)], rows_v)
        _slot_chunk(c0_hbm, base, ch, 8, c_v, pb_ref, idx0_v)
        cp0 = pltpu.async_copy(rows_v, xs_out.at[idx0_v], sem)
        _slot_chunk(c1_hbm, base, ch, 8, c_v, pb_ref, idx1_v)
        cp1 = pltpu.async_copy(rows_v, xs_out.at[idx1_v], sem)
        cp0.wait()
        cp1.wait()


def _dispatch(xf, c0, c1, pb16, nslot):
    n, c = xf.shape
    nw = 32
    tpw = n // nw
    ch = min(128, tpw)
    mesh = plsc.VectorSubcoreMesh(core_axis_name="c", subcore_axis_name="s")
    f = pl.kernel(
        functools.partial(_dispatch_body, tpw, ch),
        mesh=mesh,
        out_type=jax.ShapeDtypeStruct((nslot, c), jnp.float32),
        scratch_types=[
            pltpu.VMEM((ch, c), jnp.float32),
            pltpu.VMEM((ch,), jnp.int32),
            pltpu.VMEM((ch,), jnp.int32),
            pltpu.VMEM((ch,), jnp.int32),
            pltpu.VMEM((8, 16), jnp.int32),
            pltpu.SemaphoreType.DMA,
        ],
    )
    return f(xf, c0, c1, pb16)


def _gemm_kernel(meta_ref, xs_ref, b1_ref, b2_ref, w1_hbm, w2_hbm, out_ref,
                 w1_buf, w2_buf, sem1, sem2, *, nt):
    i = pl.program_id(0)
    e = meta_ref[i]
    valid = meta_ref[nt + i]
    par = meta_ref[2 * nt + i]
    nxt = meta_ref[3 * nt + i]
    prev = meta_ref[jnp.maximum(i - 1, 0)]
    first = jnp.logical_or(i == 0, prev != e)

    def _issue(expert, slot):
        pltpu.make_async_copy(w1_hbm.at[expert], w1_buf.at[slot],
                              sem1.at[slot]).start()
        pltpu.make_async_copy(w2_hbm.at[expert], w2_buf.at[slot],
                              sem2.at[slot]).start()

    @pl.when(i == 0)
    def _prologue():
        _issue(e, par)

    @pl.when(first)
    def _run_start():
        @pl.when(nxt >= 0)
        def _prefetch_next():
            _issue(nxt, 1 - par)

        pltpu.make_async_copy(w1_hbm.at[e], w1_buf.at[par],
                              sem1.at[par]).wait()
        pltpu.make_async_copy(w2_hbm.at[e], w2_buf.at[par],
                              sem2.at[par]).wait()

    @pl.when(valid == 1)
    def _compute():
        h = _gelu(jnp.dot(xs_ref[...], w1_buf[par],
                          preferred_element_type=jnp.float32) + b1_ref[0])
        out_ref[...] = (jnp.dot(h, w2_buf[par],
                                preferred_element_type=jnp.float32)
                        + b2_ref[0])


def _grouped_gemm(meta, xs, W1, b1, W2, b2, nt):
    nslot, c = xs.shape
    e, _, h = W1.shape
    grid_spec = pltpu.PrefetchScalarGridSpec(
        num_scalar_prefetch=1,
        grid=(nt,),
        in_specs=[
            pl.BlockSpec((TM, c), lambda i, m: (i, 0)),
            pl.BlockSpec((1, 1, h), lambda i, m: (m[i], 0, 0)),
            pl.BlockSpec((1, 1, c), lambda i, m: (m[i], 0, 0)),
            pl.BlockSpec(memory_space=pl.ANY),
            pl.BlockSpec(memory_space=pl.ANY),
        ],
        out_specs=pl.BlockSpec((TM, c), lambda i, m: (i, 0)),
        scratch_shapes=[
            pltpu.VMEM((2, c, h), jnp.float32),
            pltpu.VMEM((2, h, c), jnp.float32),
            pltpu.SemaphoreType.DMA((2,)),
            pltpu.SemaphoreType.DMA((2,)),
        ],
    )
    return pl.pallas_call(
        functools.partial(_gemm_kernel, nt=nt),
        grid_spec=grid_spec,
        out_shape=jax.ShapeDtypeStruct((nslot, c), jnp.float32),
        compiler_params=pltpu.CompilerParams(
            vmem_limit_bytes=100 * 1024 * 1024),
    )(meta, xs, b1.reshape(e, 1, h), b2.reshape(e, 1, c), W1, W2)


def _gather2_body(tpw, ch, ys_hbm, c0_hbm, c1_hbm,
                  pb_hbm, z0_hbm, z1_hbm, i0_v, i1_v, y0_v, y1_v, c_v, pb_ref,
                  sem):
    wid = lax.axis_index("s") * 2 + lax.axis_index("c")
    _slot_prelude(pb_hbm, pb_ref)
    for c in range(tpw // ch):
        base = pl.multiple_of(wid * tpw + c * ch, ch)
        _slot_chunk(c0_hbm, base, ch, 8, c_v, pb_ref, i0_v)
        cp0 = pltpu.async_copy(ys_hbm.at[i0_v], y0_v, sem)
        _slot_chunk(c1_hbm, base, ch, 8, c_v, pb_ref, i1_v)
        cp1 = pltpu.async_copy(ys_hbm.at[i1_v], y1_v, sem)
        cp0.wait()
        pltpu.sync_copy(y0_v, z0_hbm.at[pl.ds(base, ch)])
        cp1.wait()
        pltpu.sync_copy(y1_v, z1_hbm.at[pl.ds(base, ch)])


def _gather2(ys, c0, c1, pb16, n):
    nslot, c = ys.shape
    nw = 32
    tpw = n // nw
    ch = min(64, tpw)
    mesh = plsc.VectorSubcoreMesh(core_axis_name="c", subcore_axis_name="s")
    f = pl.kernel(
        functools.partial(_gather2_body, tpw, ch),
        mesh=mesh,
        out_type=(jax.ShapeDtypeStruct((n, c), jnp.float32),
                  jax.ShapeDtypeStruct((n, c), jnp.float32)),
        scratch_types=[
            pltpu.VMEM((ch,), jnp.int32),
            pltpu.VMEM((ch,), jnp.int32),
            pltpu.VMEM((ch, c), jnp.float32),
            pltpu.VMEM((ch, c), jnp.float32),
            pltpu.VMEM((ch,), jnp.int32),
            pltpu.VMEM((8, 16), jnp.int32),
            pltpu.SemaphoreType.DMA,
        ],
    )
    return f(ys, c0, c1, pb16)


def _blend_kernel(z0_ref, z1_ref, g0_ref, g1_ref, out_ref):
    out_ref[...] = g0_ref[...] * z0_ref[...] + g1_ref[...] * z1_ref[...]


def _blend(z0, z1, g0, g1):
    n, c = z0.shape
    bn = min(n, 1024)
    return pl.pallas_call(
        _blend_kernel,
        grid=(n // bn,),
        in_specs=[
            pl.BlockSpec((bn, c), lambda b: (b, 0)),
            pl.BlockSpec((bn, c), lambda b: (b, 0)),
            pl.BlockSpec((bn, 1), lambda b: (b, 0)),
            pl.BlockSpec((bn, 1), lambda b: (b, 0)),
        ],
        out_specs=pl.BlockSpec((bn, c), lambda b: (b, 0)),
        out_shape=jax.ShapeDtypeStruct((n, c), jnp.float32),
    )(z0, z1, g0, g1)


def kernel(x, Wg, W1, b1, W2, b2):
    Bx, Tx, C = x.shape
    E = Wg.shape[1]
    N = Bx * Tx
    nt = (2 * N) // TM + E
    nslot = nt * TM
    xf = x.reshape(N, C)

    g0, g1, c0, c1, pb16, meta = _router(xf, Wg, nt)
    c0 = c0.reshape(N)
    c1 = c1.reshape(N)
    meta = meta.reshape(4 * nt)

    xs = _dispatch(xf, c0, c1, pb16, nslot)
    ys = _grouped_gemm(meta, xs, W1, b1, W2, b2, nt)
    z0, z1 = _gather2(ys, c0, c1, pb16, N)
    outf = _blend(z0, z1, g0, g1)
    return outf.reshape(Bx, Tx, C)
